# baseline (speedup 1.0000x reference)
import functools

import jax
import jax.numpy as jnp
from jax import lax
from jax.experimental import pallas as pl
from jax.experimental.pallas import tpu as pltpu
from jax.experimental.pallas import tpu_sc as plsc

B = 256
WENC = 4096
WDEC = 4096
MAXS = 2048
NC = 2
NS = 16
NW = NC * NS
ROWS = B // NW
BUF = WENC + 2 * MAXS
L = 16
CHUNKS = WENC // L
OSLOTS = 4
YD_BR = 32


def _frames_sc(xe, xe_lens, xt, xt_lens, prev):
    mesh = plsc.VectorSubcoreMesh(core_axis_name="c", subcore_axis_name="s")

    @functools.partial(
        pl.kernel,
        mesh=mesh,
        compiler_params=pltpu.CompilerParams(needs_layout_passes=False),
        out_type=[
            jax.ShapeDtypeStruct((B, WENC), jnp.int32),
            jax.ShapeDtypeStruct((B, WENC), jnp.int32),
        ],
        scratch_types=(
            [pltpu.VMEM((BUF,), jnp.int32) for _ in range(ROWS)]
            + [pltpu.VMEM((WENC,), jnp.int32) for _ in range(OSLOTS)]
            + [pltpu.VMEM((WENC,), jnp.int32) for _ in range(OSLOTS)]
            + [
                pltpu.VMEM((L,), jnp.int32),
                pltpu.VMEM((L,), jnp.int32),
                pltpu.SemaphoreType.DMA,
            ]
            + [pltpu.SemaphoreType.DMA for _ in range(ROWS)]
            + [pltpu.SemaphoreType.DMA for _ in range(OSLOTS)]
        ),
    )
    def k(xe_h, el_h, xt_h, tl_h, prev_h, ye_h, p_h, *scr):
        bufs = scr[:ROWS]
        yes = scr[ROWS:ROWS + OSLOTS]
        ps = scr[ROWS + OSLOTS:ROWS + 2 * OSLOTS]
        n = ROWS + 2 * OSLOTS
        el_v, tl_v, sem_lens = scr[n:n + 3]
        sem_in = scr[n + 3:n + 3 + ROWS]
        sem_out = scr[n + 3 + ROWS:]

        wid = lax.axis_index("s") * NC + lax.axis_index("c")
        base = wid * ROWS
        lanes = lax.iota(jnp.int32, L)

        lens_handles = (
            pltpu.async_copy(el_h.at[pl.ds(base, ROWS)],
                             el_v.at[pl.ds(0, ROWS)], sem_lens),
            pltpu.async_copy(tl_h.at[pl.ds(base, ROWS)],
                             tl_v.at[pl.ds(0, ROWS)], sem_lens),
        )
        in_flight = []
        for r in range(ROWS):
            buf = bufs[r]
            row = base + r
            s = sem_in[r]
            in_flight.append((
                pltpu.async_copy(prev_h.at[row], buf.at[pl.ds(0, WENC)], s),
                pltpu.async_copy(xe_h.at[row], buf.at[pl.ds(WENC, MAXS)], s),
                pltpu.async_copy(xt_h.at[row],
                                 buf.at[pl.ds(WENC + MAXS, MAXS)], s),
            ))

        for h in lens_handles:
            h.wait()
        el_vec = el_v[...]
        tl_vec = tl_v[...]

        out_flight = {}
        for r in range(ROWS):
            slot = r % OSLOTS
            row = base + r
            if r - OSLOTS in out_flight:
                for h in out_flight.pop(r - OSLOTS):
                    h.wait()
            for h in in_flight[r]:
                h.wait()

            el = el_vec[r]
            tl = tl_vec[r]
            el_lanes = el + lanes
            tl_lanes = tl + lanes
            a_vec = tl_lanes + el
            b_vec = tl_lanes + MAXS
            buf = bufs[r]
            yev = yes[slot]
            pv = ps[slot]

            @pl.loop(0, CHUNKS, unroll=2)
            def _(kk):
                off = pl.multiple_of(kk * L, L)
                yev[pl.ds(off, L)] = plsc.load_gather(buf, [el_lanes + off])
                q = tl_lanes + off
                idx2 = jnp.where(q < WENC, a_vec + off, b_vec + off)
                pv[pl.ds(off, L)] = plsc.load_gather(buf, [idx2])

            out_flight[r] = (
                pltpu.async_copy(yev, ye_h.at[row], sem_out[slot]),
                pltpu.async_copy(pv, p_h.at[row], sem_out[slot]),
            )

        for r, hs in sorted(out_flight.items()):
            for h in hs:
                h.wait()

    return k(xe, xe_lens, xt, xt_lens, prev)


def _pad_tc(xd):
    def body(x_ref, o_ref):
        o_ref[:, :MAXS] = x_ref[...]
        o_ref[:, MAXS:] = jnp.zeros((YD_BR, WDEC - MAXS), jnp.int32)

    return pl.pallas_call(
        body,
        grid=(B // YD_BR,),
        in_specs=[pl.BlockSpec((YD_BR, MAXS), lambda i: (i, 0))],
        out_specs=pl.BlockSpec((YD_BR, WDEC), lambda i: (i, 0)),
        out_shape=jax.ShapeDtypeStruct((B, WDEC), jnp.int32),
    )(xd)


def kernel(xe, xe_lens, xd, xd_lens, xt, xt_lens, prev):
    el = xe_lens.astype(jnp.int32)
    dl = xd_lens.astype(jnp.int32)
    ye, p = _frames_sc(xe, el, xt, xt_lens.astype(jnp.int32), prev)
    yd = _pad_tc(xd)
    return (ye, el, yd, dl, p)
